# baseline (speedup 1.0000x reference)
import functools

import jax
import jax.numpy as jnp
from jax import lax
from jax.experimental import pallas as pl
from jax.experimental.pallas import tpu as pltpu
from jax.experimental.pallas import tpu_sc as plsc

VOCAB = 8192
D = 64
NTOK = 8 * 576


TT = 1152
KB = 8192
N_K = VOCAB // KB


def _argmax_body(x_ref, w_ref, idx_ref, max_s, idx_s):
    k = pl.program_id(1)
    logits = jax.lax.dot_general(
        x_ref[...], w_ref[...],
        dimension_numbers=(((1,), (1,)), ((), ())),
        preferred_element_type=jnp.float32,
    )
    local_max = jnp.max(logits, axis=1, keepdims=True)
    col = lax.broadcasted_iota(jnp.int32, (TT, KB), 1).astype(jnp.float32)
    local_arg = jnp.min(
        jnp.where(logits == local_max, col, float(VOCAB)), axis=1, keepdims=True
    ).astype(jnp.int32) + k * KB

    @pl.when(k == 0)
    def _init():
        max_s[...] = local_max
        idx_s[...] = local_arg

    @pl.when(k > 0)
    def _combine():
        better = local_max > max_s[...]
        idx_s[...] = jnp.where(better, local_arg, idx_s[...])
        max_s[...] = jnp.maximum(max_s[...], local_max)

    @pl.when(k == N_K - 1)
    def _emit():
        idx_ref[...] = idx_s[...]


def _tc_argmax(x, W):
    ntok = x.shape[0]
    return pl.pallas_call(
        _argmax_body,
        grid=(ntok // TT, N_K),
        in_specs=[
            pl.BlockSpec((TT, D), lambda t, k: (t, 0)),
            pl.BlockSpec((KB, D), lambda t, k: (k, 0)),
        ],
        out_specs=pl.BlockSpec((TT, 1), lambda t, k: (t, 0)),
        out_shape=jax.ShapeDtypeStruct((ntok, 1), jnp.int32),
        scratch_shapes=[
            pltpu.VMEM((TT, 1), jnp.float32),
            pltpu.VMEM((TT, 1), jnp.int32),
        ],
        compiler_params=pltpu.CompilerParams(
            dimension_semantics=("arbitrary", "arbitrary"),
        ),
    )(x, W)


_NW = 32


@functools.lru_cache(maxsize=2)
def _sc_gather_fn(ntok):
    bpw = ntok // _NW
    mesh = plsc.VectorSubcoreMesh(core_axis_name="c", subcore_axis_name="s")

    @functools.partial(
        pl.kernel,
        out_type=jax.ShapeDtypeStruct((ntok, D), jnp.float32),
        mesh=mesh,
        scratch_types=[
            pltpu.VMEM((bpw,), jnp.int32),
            pltpu.VMEM((bpw, D), jnp.float32),
            pltpu.SemaphoreType.DMA,
        ],
        compiler_params=pltpu.CompilerParams(use_tc_tiling_on_sc=False),
    )
    def _sc_gather(table_hbm, idx_hbm, out_hbm, idx_v, rows_v, sem):
        wid = lax.axis_index("s") * 2 + lax.axis_index("c")
        base = wid * bpw
        pltpu.sync_copy(idx_hbm.at[pl.ds(base, bpw)], idx_v)
        pltpu.async_copy(table_hbm.at[idx_v], rows_v, sem).wait()
        pltpu.sync_copy(rows_v, out_hbm.at[pl.ds(base, bpw)])

    return _sc_gather


def kernel(input, W):
    x = input.reshape(NTOK, D)
    half = NTOK // 2
    gather = _sc_gather_fn(half)
    idx1 = _tc_argmax(x[:half], W).reshape(half)
    q1 = gather(W, idx1)
    idx2 = _tc_argmax(x[half:], W).reshape(half)
    q2 = gather(W, idx2)
    quantize = jnp.concatenate([q1, q2], axis=0)
    idxs = jnp.concatenate([idx1, idx2], axis=0)
    return quantize.reshape(8, 576, D), idxs.reshape(8, 576)
